# Initial kernel scaffold; baseline (speedup 1.0000x reference)
import jax
import jax.numpy as jnp
from jax.experimental import pallas as pl


def kernel(h, coord, edge_index, edge_attr, W1, b1, W2, b2, W3, b3, W4, b4):
    raise NotImplementedError("write your pallas kernel here")


# speedup vs baseline: 2.4569x; 2.4569x over previous
import functools

import jax
import jax.numpy as jnp
from jax import lax
from jax.experimental import pallas as pl
from jax.experimental.pallas import tpu as pltpu
from jax.experimental.pallas import tpu_sc as plsc

N = 10000
E = 320000
F = 128

NC = 2
NS = 16
NW = NC * NS
PER_W = E // NW
K = 80
ITERS = PER_W // K
NPAD = 10240
ROWS_PER_TILE = NPAD // NS

BN = 1000
BE = 2560


def _node_pre_body(h_ref, c16_ref, w1at_ref, w1bt_ref, b1_ref, w1r_ref,
                   a_ref, b_ref):
    h = h_ref[...]
    c16 = c16_ref[...]
    c2 = jnp.sum(c16 * c16, axis=1, keepdims=True)
    rad = c2 * w1r_ref[...]
    a_ref[...] = jnp.dot(h, w1at_ref[...],
                         preferred_element_type=jnp.float32) + b1_ref[...] + rad
    b_ref[...] = jnp.dot(h, w1bt_ref[...],
                         preferred_element_type=jnp.float32) + rad


def _node_pre(h, c16, w1at, w1bt, b1, w1r):
    grid = (N // BN,)
    return pl.pallas_call(
        _node_pre_body,
        grid=grid,
        in_specs=[
            pl.BlockSpec((BN, F), lambda i: (i, 0)),
            pl.BlockSpec((BN, 16), lambda i: (i, 0)),
            pl.BlockSpec((F, F), lambda i: (0, 0)),
            pl.BlockSpec((F, F), lambda i: (0, 0)),
            pl.BlockSpec((1, F), lambda i: (0, 0)),
            pl.BlockSpec((1, F), lambda i: (0, 0)),
        ],
        out_specs=[
            pl.BlockSpec((BN, F), lambda i: (i, 0)),
            pl.BlockSpec((BN, F), lambda i: (i, 0)),
        ],
        out_shape=[
            jax.ShapeDtypeStruct((N, F), jnp.float32),
            jax.ShapeDtypeStruct((N, F), jnp.float32),
        ],
    )(h, c16, w1at, w1bt, b1, w1r)


def _gather_body(ta, tb, c16, row_hbm, col_hbm,
                 g1_hbm, g2_hbm, cr_hbm, cc_hbm,
                 rowv, colv, av, bv, crv, ccv, sem):
    wid = lax.axis_index("s") * NC + lax.axis_index("c")
    base = wid * PER_W

    def body(i, carry):
        off = pl.multiple_of(base + i * K, 8)
        pltpu.sync_copy(row_hbm.at[pl.ds(off, K)], rowv)
        pltpu.sync_copy(col_hbm.at[pl.ds(off, K)], colv)
        cp1 = pltpu.async_copy(ta.at[rowv], av, sem)
        cp2 = pltpu.async_copy(tb.at[colv], bv, sem)
        cp3 = pltpu.async_copy(c16.at[rowv], crv, sem)
        cp4 = pltpu.async_copy(c16.at[colv], ccv, sem)
        cp1.wait()
        cp2.wait()
        cp3.wait()
        cp4.wait()
        pltpu.sync_copy(av, g1_hbm.at[pl.ds(off, K)])
        pltpu.sync_copy(bv, g2_hbm.at[pl.ds(off, K)])
        pltpu.sync_copy(crv, cr_hbm.at[pl.ds(off, K)])
        pltpu.sync_copy(ccv, cc_hbm.at[pl.ds(off, K)])
        return carry

    lax.fori_loop(0, ITERS, body, 0)


def _sc_gather(ta, tb, c16, row, col):
    mesh = plsc.VectorSubcoreMesh(core_axis_name="c", subcore_axis_name="s")
    kfn = pl.kernel(
        _gather_body,
        out_type=[
            jax.ShapeDtypeStruct((E, F), jnp.float32),
            jax.ShapeDtypeStruct((E, F), jnp.float32),
            jax.ShapeDtypeStruct((E, 16), jnp.float32),
            jax.ShapeDtypeStruct((E, 16), jnp.float32),
        ],
        mesh=mesh,
        scratch_types=[
            pltpu.VMEM((K,), jnp.int32),
            pltpu.VMEM((K,), jnp.int32),
            pltpu.VMEM((K, F), jnp.float32),
            pltpu.VMEM((K, F), jnp.float32),
            pltpu.VMEM((K, 16), jnp.float32),
            pltpu.VMEM((K, 16), jnp.float32),
            pltpu.SemaphoreType.DMA,
        ],
        compiler_params=pltpu.CompilerParams(use_tc_tiling_on_sc=False),
    )
    return kfn(ta, tb, c16, row, col)


def _edge_body(g1_ref, g2_ref, cr_ref, cc_ref, ea_ref,
               w1et_ref, w2t_ref, b2_ref, w1r_ref, y_ref):
    cross = jnp.sum(cr_ref[...] * cc_ref[...], axis=1, keepdims=True)
    pre1 = (g1_ref[...] + g2_ref[...]
            - 2.0 * cross * w1r_ref[...]
            + jnp.dot(ea_ref[...], w1et_ref[...],
                      preferred_element_type=jnp.float32))
    m = pre1 * jax.nn.sigmoid(pre1)
    pre2 = jnp.dot(m, w2t_ref[...],
                   preferred_element_type=jnp.float32) + b2_ref[...]
    y_ref[...] = pre2 * jax.nn.sigmoid(pre2)


def _edge_mlp(g1, g2, cr, cc, ea8, w1et8, w2t, b2, w1r):
    grid = (E // BE,)
    return pl.pallas_call(
        _edge_body,
        grid=grid,
        in_specs=[
            pl.BlockSpec((BE, F), lambda i: (i, 0)),
            pl.BlockSpec((BE, F), lambda i: (i, 0)),
            pl.BlockSpec((BE, 16), lambda i: (i, 0)),
            pl.BlockSpec((BE, 16), lambda i: (i, 0)),
            pl.BlockSpec((BE, 8), lambda i: (i, 0)),
            pl.BlockSpec((8, F), lambda i: (0, 0)),
            pl.BlockSpec((F, F), lambda i: (0, 0)),
            pl.BlockSpec((1, F), lambda i: (0, 0)),
            pl.BlockSpec((1, F), lambda i: (0, 0)),
        ],
        out_specs=pl.BlockSpec((BE, F), lambda i: (i, 0)),
        out_shape=jax.ShapeDtypeStruct((E, F), jnp.float32),
    )(g1, g2, cr, cc, ea8, w1et8, w2t, b2, w1r)


HALF = 5120
ACC_ROWS = HALF + 128
ZROWS = ACC_ROWS // NS
PER_TILE_E = E // NS
SC_ITERS = PER_TILE_E // K
DUMP_ROWS = HALF // NS


def _scatter_body(y_hbm, row_hbm, zeros_hbm, out_hbm,
                  idxv, idx2v, yv, dumpv, acc):
    c = lax.axis_index("c")
    s = lax.axis_index("s")
    base = s * PER_TILE_E
    lo = c * HALF

    pltpu.sync_copy(zeros_hbm, acc.at[pl.ds(s * ZROWS, ZROWS)])
    plsc.subcore_barrier()

    def body(i, carry):
        off = pl.multiple_of(base + i * K, 8)
        pltpu.sync_copy(row_hbm.at[pl.ds(off, K)], idxv)
        pltpu.sync_copy(y_hbm.at[pl.ds(off, K)], yv)
        for j in range(K // 16):
            v = idxv[pl.ds(j * 16, 16)]
            v2 = v - lo
            in_half = (v2 >= 0) & (v2 < HALF)
            dummy = HALF + (v & 127)
            idx2v[pl.ds(j * 16, 16)] = jnp.where(in_half, v2, dummy)
        pltpu.sync_copy(yv, acc.at[idx2v], add=True)
        return carry

    lax.fori_loop(0, SC_ITERS, body, 0)
    plsc.subcore_barrier()

    pltpu.sync_copy(acc.at[pl.ds(s * DUMP_ROWS, DUMP_ROWS)], dumpv)
    pltpu.sync_copy(dumpv, out_hbm.at[c].at[pl.ds(s * DUMP_ROWS, DUMP_ROWS)])


def _sc_scatter(y, row, zeros_nf):
    mesh = plsc.VectorSubcoreMesh(core_axis_name="c", subcore_axis_name="s")
    kfn = pl.kernel(
        _scatter_body,
        out_type=jax.ShapeDtypeStruct((NC, HALF, F), jnp.float32),
        mesh=mesh,
        scratch_types=[
            pltpu.VMEM((K,), jnp.int32),
            pltpu.VMEM((K,), jnp.int32),
            pltpu.VMEM((K, F), jnp.float32),
            pltpu.VMEM((DUMP_ROWS, F), jnp.float32),
            pltpu.VMEM_SHARED((ACC_ROWS, F), jnp.float32),
        ],
    )
    return kfn(y, row, zeros_nf)


def _node_body(h_ref, agg_ref, w3at_ref, w3bt_ref, b3_ref,
               w4t_ref, b4_ref, out_ref):
    h = h_ref[...]
    agg = agg_ref[...]
    pre = (jnp.dot(h, w3at_ref[...], preferred_element_type=jnp.float32)
           + jnp.dot(agg, w3bt_ref[...], preferred_element_type=jnp.float32)
           + b3_ref[...])
    n = pre * jax.nn.sigmoid(pre)
    out_ref[...] = h + jnp.dot(n, w4t_ref[...],
                               preferred_element_type=jnp.float32) + b4_ref[...]


def _node_mlp(h, agg, w3at, w3bt, b3, w4t, b4):
    grid = (N // BN,)
    return pl.pallas_call(
        _node_body,
        grid=grid,
        in_specs=[
            pl.BlockSpec((BN, F), lambda i: (i, 0)),
            pl.BlockSpec((BN, F), lambda i: (i, 0)),
            pl.BlockSpec((F, F), lambda i: (0, 0)),
            pl.BlockSpec((F, F), lambda i: (0, 0)),
            pl.BlockSpec((1, F), lambda i: (0, 0)),
            pl.BlockSpec((F, F), lambda i: (0, 0)),
            pl.BlockSpec((1, F), lambda i: (0, 0)),
        ],
        out_specs=pl.BlockSpec((BN, F), lambda i: (i, 0)),
        out_shape=jax.ShapeDtypeStruct((N, F), jnp.float32),
    )(h, agg, w3at, w3bt, b3, w4t, b4)


def kernel(h, coord, edge_index, edge_attr, W1, b1, W2, b2, W3, b3, W4, b4):
    row = edge_index[0].astype(jnp.int32)
    col = edge_index[1].astype(jnp.int32)

    w1at = W1[:, :F].T
    w1bt = W1[:, F:2 * F].T
    w1r = W1[:, 2 * F].reshape(1, F)
    w1et8 = jnp.zeros((8, F), jnp.float32).at[:4, :].set(W1[:, 2 * F + 1:].T)
    b1r = b1.reshape(1, F)
    b2r = b2.reshape(1, F)
    w3at = W3[:, :F].T
    w3bt = W3[:, F:].T
    b3r = b3.reshape(1, F)
    w4t = W4.T
    b4r = b4.reshape(1, F)

    c16 = jnp.zeros((N, 16), jnp.float32).at[:, :3].set(coord)
    ea8 = jnp.zeros((E, 8), jnp.float32).at[:, :4].set(edge_attr)

    a_tab, b_tab = _node_pre(h, c16, w1at, w1bt, b1r, w1r)
    g1, g2, cr, cc = _sc_gather(a_tab, b_tab, c16, row, col)
    y = _edge_mlp(g1, g2, cr, cc, ea8, w1et8, w2t=W2.T, b2=b2r, w1r=w1r)
    zeros_nf = jnp.zeros((ZROWS, F), jnp.float32)
    partials = _sc_scatter(y, row, zeros_nf)
    agg = partials.reshape(NC * HALF, F)[:N]
    out = _node_mlp(h, agg, w3at, w3bt, b3r, w4t, b4r)
    return (out, coord)


# speedup vs baseline: 3.0376x; 1.2364x over previous
import functools

import jax
import jax.numpy as jnp
from jax import lax
from jax.experimental import pallas as pl
from jax.experimental.pallas import tpu as pltpu
from jax.experimental.pallas import tpu_sc as plsc

N = 10000
E = 320000
F = 128

NC = 2
NS = 16
NW = NC * NS
PER_W = E // NW
K = 80
ITERS = PER_W // K
NPAD = 10240
ROWS_PER_TILE = NPAD // NS

BN = 1000
BE = 2560


def _node_pre_body(h_ref, c16_ref, w1at_ref, w1bt_ref, b1_ref, w1r_ref,
                   a_ref, b_ref):
    h = h_ref[...]
    c16 = c16_ref[...]
    c2 = jnp.sum(c16 * c16, axis=1, keepdims=True)
    rad = c2 * w1r_ref[...]
    a_ref[...] = jnp.dot(h, w1at_ref[...],
                         preferred_element_type=jnp.float32) + b1_ref[...] + rad
    b_ref[...] = jnp.dot(h, w1bt_ref[...],
                         preferred_element_type=jnp.float32) + rad


def _node_pre(h, c16, w1at, w1bt, b1, w1r):
    grid = (N // BN,)
    return pl.pallas_call(
        _node_pre_body,
        grid=grid,
        in_specs=[
            pl.BlockSpec((BN, F), lambda i: (i, 0)),
            pl.BlockSpec((BN, 16), lambda i: (i, 0)),
            pl.BlockSpec((F, F), lambda i: (0, 0)),
            pl.BlockSpec((F, F), lambda i: (0, 0)),
            pl.BlockSpec((1, F), lambda i: (0, 0)),
            pl.BlockSpec((1, F), lambda i: (0, 0)),
        ],
        out_specs=[
            pl.BlockSpec((BN, F), lambda i: (i, 0)),
            pl.BlockSpec((BN, F), lambda i: (i, 0)),
        ],
        out_shape=[
            jax.ShapeDtypeStruct((N, F), jnp.float32),
            jax.ShapeDtypeStruct((N, F), jnp.float32),
        ],
    )(h, c16, w1at, w1bt, b1, w1r)


def _gather_body(ta, tb, c16, row_hbm, col_hbm,
                 g1_hbm, g2_hbm, cr_hbm, cc_hbm,
                 rowall, colall,
                 av0, bv0, crv0, ccv0, av1, bv1, crv1, ccv1,
                 gsem0, gsem1, wsem0, wsem1):
    wid = lax.axis_index("s") * NC + lax.axis_index("c")
    base = pl.multiple_of(wid * PER_W, 8)

    pltpu.sync_copy(row_hbm.at[pl.ds(base, PER_W)], rowall)
    pltpu.sync_copy(col_hbm.at[pl.ds(base, PER_W)], colall)

    def start_g(i, av, bv, crv, ccv, sem):
        o = pl.multiple_of(i * K, 8)
        idxr = rowall.at[pl.ds(o, K)]
        idxc = colall.at[pl.ds(o, K)]
        pltpu.async_copy(ta.at[idxr], av, sem)
        pltpu.async_copy(tb.at[idxc], bv, sem)
        pltpu.async_copy(c16.at[idxr], crv, sem)
        pltpu.async_copy(c16.at[idxc], ccv, sem)

    def wait_g(av, bv, crv, ccv, sem):
        idxr = rowall.at[pl.ds(0, K)]
        idxc = colall.at[pl.ds(0, K)]
        pltpu.make_async_copy(ta.at[idxr], av, sem).wait()
        pltpu.make_async_copy(tb.at[idxc], bv, sem).wait()
        pltpu.make_async_copy(c16.at[idxr], crv, sem).wait()
        pltpu.make_async_copy(c16.at[idxc], ccv, sem).wait()

    def start_w(i, av, bv, crv, ccv, sem):
        off = pl.multiple_of(base + i * K, 8)
        pltpu.async_copy(av, g1_hbm.at[pl.ds(off, K)], sem)
        pltpu.async_copy(bv, g2_hbm.at[pl.ds(off, K)], sem)
        pltpu.async_copy(crv, cr_hbm.at[pl.ds(off, K)], sem)
        pltpu.async_copy(ccv, cc_hbm.at[pl.ds(off, K)], sem)

    def wait_w(av, bv, crv, ccv, sem):
        pltpu.make_async_copy(av, g1_hbm.at[pl.ds(0, K)], sem).wait()
        pltpu.make_async_copy(bv, g2_hbm.at[pl.ds(0, K)], sem).wait()
        pltpu.make_async_copy(crv, cr_hbm.at[pl.ds(0, K)], sem).wait()
        pltpu.make_async_copy(ccv, cc_hbm.at[pl.ds(0, K)], sem).wait()

    start_g(0, av0, bv0, crv0, ccv0, gsem0)

    def body(t, carry):
        c0 = 2 * t

        @pl.when(t > 0)
        def _():
            wait_w(av1, bv1, crv1, ccv1, wsem1)

        start_g(c0 + 1, av1, bv1, crv1, ccv1, gsem1)
        wait_g(av0, bv0, crv0, ccv0, gsem0)
        start_w(c0, av0, bv0, crv0, ccv0, wsem0)
        wait_w(av0, bv0, crv0, ccv0, wsem0)

        @pl.when(c0 + 2 < ITERS)
        def _():
            start_g(c0 + 2, av0, bv0, crv0, ccv0, gsem0)

        wait_g(av1, bv1, crv1, ccv1, gsem1)
        start_w(c0 + 1, av1, bv1, crv1, ccv1, wsem1)
        return carry

    lax.fori_loop(0, ITERS // 2, body, 0)

    wait_w(av1, bv1, crv1, ccv1, wsem1)
    wait_g(av0, bv0, crv0, ccv0, gsem0)
    start_w(ITERS - 1, av0, bv0, crv0, ccv0, wsem0)
    wait_w(av0, bv0, crv0, ccv0, wsem0)


def _sc_gather(ta, tb, c16, row, col):
    mesh = plsc.VectorSubcoreMesh(core_axis_name="c", subcore_axis_name="s")
    kfn = pl.kernel(
        _gather_body,
        out_type=[
            jax.ShapeDtypeStruct((E, F), jnp.float32),
            jax.ShapeDtypeStruct((E, F), jnp.float32),
            jax.ShapeDtypeStruct((E, 16), jnp.float32),
            jax.ShapeDtypeStruct((E, 16), jnp.float32),
        ],
        mesh=mesh,
        scratch_types=[
            pltpu.VMEM((PER_W,), jnp.int32),
            pltpu.VMEM((PER_W,), jnp.int32),
            pltpu.VMEM((K, F), jnp.float32),
            pltpu.VMEM((K, F), jnp.float32),
            pltpu.VMEM((K, 16), jnp.float32),
            pltpu.VMEM((K, 16), jnp.float32),
            pltpu.VMEM((K, F), jnp.float32),
            pltpu.VMEM((K, F), jnp.float32),
            pltpu.VMEM((K, 16), jnp.float32),
            pltpu.VMEM((K, 16), jnp.float32),
            pltpu.SemaphoreType.DMA,
            pltpu.SemaphoreType.DMA,
            pltpu.SemaphoreType.DMA,
            pltpu.SemaphoreType.DMA,
        ],
        compiler_params=pltpu.CompilerParams(use_tc_tiling_on_sc=False),
    )
    return kfn(ta, tb, c16, row, col)


def _edge_body(g1_ref, g2_ref, cr_ref, cc_ref, ea_ref,
               w1et_ref, w2t_ref, b2_ref, w1r_ref, y_ref):
    cross = jnp.sum(cr_ref[...] * cc_ref[...], axis=1, keepdims=True)
    pre1 = (g1_ref[...] + g2_ref[...]
            - 2.0 * cross * w1r_ref[...]
            + jnp.dot(ea_ref[...], w1et_ref[...],
                      preferred_element_type=jnp.float32))
    m = pre1 * jax.nn.sigmoid(pre1)
    pre2 = jnp.dot(m, w2t_ref[...],
                   preferred_element_type=jnp.float32) + b2_ref[...]
    y_ref[...] = pre2 * jax.nn.sigmoid(pre2)


def _edge_mlp(g1, g2, cr, cc, ea8, w1et8, w2t, b2, w1r):
    grid = (E // BE,)
    return pl.pallas_call(
        _edge_body,
        grid=grid,
        in_specs=[
            pl.BlockSpec((BE, F), lambda i: (i, 0)),
            pl.BlockSpec((BE, F), lambda i: (i, 0)),
            pl.BlockSpec((BE, 16), lambda i: (i, 0)),
            pl.BlockSpec((BE, 16), lambda i: (i, 0)),
            pl.BlockSpec((BE, 8), lambda i: (i, 0)),
            pl.BlockSpec((8, F), lambda i: (0, 0)),
            pl.BlockSpec((F, F), lambda i: (0, 0)),
            pl.BlockSpec((1, F), lambda i: (0, 0)),
            pl.BlockSpec((1, F), lambda i: (0, 0)),
        ],
        out_specs=pl.BlockSpec((BE, F), lambda i: (i, 0)),
        out_shape=jax.ShapeDtypeStruct((E, F), jnp.float32),
    )(g1, g2, cr, cc, ea8, w1et8, w2t, b2, w1r)


HALF = 5120
ACC_ROWS = HALF + 128
ZROWS = ACC_ROWS // NS
PER_TILE_E = E // NS
SC_ITERS = PER_TILE_E // K
DUMP_ROWS = HALF // NS


def _scatter_body(y_hbm, row_hbm, zeros_hbm, out_hbm,
                  it0, it1, ix0, ix1, yv0, yv1, dumpv, acc,
                  lsem0, lsem1, ssem0, ssem1):
    c = lax.axis_index("c")
    s = lax.axis_index("s")
    base = pl.multiple_of(s * PER_TILE_E, 8)
    lo = c * HALF

    pltpu.sync_copy(zeros_hbm, acc.at[pl.ds(s * ZROWS, ZROWS)])

    def start_l(i, it, yv, sem):
        off = pl.multiple_of(base + i * K, 8)
        pltpu.async_copy(row_hbm.at[pl.ds(off, K)], it, sem)
        pltpu.async_copy(y_hbm.at[pl.ds(off, K)], yv, sem)

    def wait_l(it, yv, sem):
        pltpu.make_async_copy(row_hbm.at[pl.ds(0, K)], it, sem).wait()
        pltpu.make_async_copy(y_hbm.at[pl.ds(0, K)], yv, sem).wait()

    def remap(it, ix):
        for j in range(K // 16):
            v = it[pl.ds(j * 16, 16)]
            v2 = v - lo
            in_half = (v2 >= 0) & (v2 < HALF)
            ix[pl.ds(j * 16, 16)] = jnp.where(in_half, v2, HALF + (v & 127))

    def start_s(ix, yv, sem):
        pltpu.async_copy(yv, acc.at[ix], sem, add=True)

    def wait_s(ix, yv, sem):
        pltpu.make_async_copy(yv, acc.at[ix], sem).wait()

    start_l(0, it0, yv0, lsem0)
    plsc.subcore_barrier()

    def body(t, carry):
        c0 = 2 * t
        wait_l(it0, yv0, lsem0)
        remap(it0, ix0)
        start_s(ix0, yv0, ssem0)

        @pl.when(t > 0)
        def _():
            wait_s(ix1, yv1, ssem1)

        start_l(c0 + 1, it1, yv1, lsem1)
        wait_l(it1, yv1, lsem1)
        remap(it1, ix1)
        start_s(ix1, yv1, ssem1)
        wait_s(ix0, yv0, ssem0)

        @pl.when(t + 1 < SC_ITERS // 2)
        def _():
            start_l(c0 + 2, it0, yv0, lsem0)

        return carry

    lax.fori_loop(0, SC_ITERS // 2, body, 0)
    wait_s(ix1, yv1, ssem1)
    plsc.subcore_barrier()

    pltpu.sync_copy(acc.at[pl.ds(s * DUMP_ROWS, DUMP_ROWS)], dumpv)
    pltpu.sync_copy(dumpv, out_hbm.at[c].at[pl.ds(s * DUMP_ROWS, DUMP_ROWS)])


def _sc_scatter(y, row, zeros_nf):
    mesh = plsc.VectorSubcoreMesh(core_axis_name="c", subcore_axis_name="s")
    kfn = pl.kernel(
        _scatter_body,
        out_type=jax.ShapeDtypeStruct((NC, HALF, F), jnp.float32),
        mesh=mesh,
        scratch_types=[
            pltpu.VMEM((K,), jnp.int32),
            pltpu.VMEM((K,), jnp.int32),
            pltpu.VMEM((K,), jnp.int32),
            pltpu.VMEM((K,), jnp.int32),
            pltpu.VMEM((K, F), jnp.float32),
            pltpu.VMEM((K, F), jnp.float32),
            pltpu.VMEM((DUMP_ROWS, F), jnp.float32),
            pltpu.VMEM_SHARED((ACC_ROWS, F), jnp.float32),
            pltpu.SemaphoreType.DMA,
            pltpu.SemaphoreType.DMA,
            pltpu.SemaphoreType.DMA,
            pltpu.SemaphoreType.DMA,
        ],
    )
    return kfn(y, row, zeros_nf)


def _node_body(h_ref, agg_ref, w3at_ref, w3bt_ref, b3_ref,
               w4t_ref, b4_ref, out_ref):
    h = h_ref[...]
    agg = agg_ref[...]
    pre = (jnp.dot(h, w3at_ref[...], preferred_element_type=jnp.float32)
           + jnp.dot(agg, w3bt_ref[...], preferred_element_type=jnp.float32)
           + b3_ref[...])
    n = pre * jax.nn.sigmoid(pre)
    out_ref[...] = h + jnp.dot(n, w4t_ref[...],
                               preferred_element_type=jnp.float32) + b4_ref[...]


def _node_mlp(h, agg, w3at, w3bt, b3, w4t, b4):
    grid = (N // BN,)
    return pl.pallas_call(
        _node_body,
        grid=grid,
        in_specs=[
            pl.BlockSpec((BN, F), lambda i: (i, 0)),
            pl.BlockSpec((BN, F), lambda i: (i, 0)),
            pl.BlockSpec((F, F), lambda i: (0, 0)),
            pl.BlockSpec((F, F), lambda i: (0, 0)),
            pl.BlockSpec((1, F), lambda i: (0, 0)),
            pl.BlockSpec((F, F), lambda i: (0, 0)),
            pl.BlockSpec((1, F), lambda i: (0, 0)),
        ],
        out_specs=pl.BlockSpec((BN, F), lambda i: (i, 0)),
        out_shape=jax.ShapeDtypeStruct((N, F), jnp.float32),
    )(h, agg, w3at, w3bt, b3, w4t, b4)


def kernel(h, coord, edge_index, edge_attr, W1, b1, W2, b2, W3, b3, W4, b4):
    row = edge_index[0].astype(jnp.int32)
    col = edge_index[1].astype(jnp.int32)

    w1at = W1[:, :F].T
    w1bt = W1[:, F:2 * F].T
    w1r = W1[:, 2 * F].reshape(1, F)
    w1et8 = jnp.zeros((8, F), jnp.float32).at[:4, :].set(W1[:, 2 * F + 1:].T)
    b1r = b1.reshape(1, F)
    b2r = b2.reshape(1, F)
    w3at = W3[:, :F].T
    w3bt = W3[:, F:].T
    b3r = b3.reshape(1, F)
    w4t = W4.T
    b4r = b4.reshape(1, F)

    c16 = jnp.zeros((N, 16), jnp.float32).at[:, :3].set(coord)
    ea8 = jnp.zeros((E, 8), jnp.float32).at[:, :4].set(edge_attr)

    a_tab, b_tab = _node_pre(h, c16, w1at, w1bt, b1r, w1r)
    g1, g2, cr, cc = _sc_gather(a_tab, b_tab, c16, row, col)
    y = _edge_mlp(g1, g2, cr, cc, ea8, w1et8, w2t=W2.T, b2=b2r, w1r=w1r)
    zeros_nf = jnp.zeros((ZROWS, F), jnp.float32)
    partials = _sc_scatter(y, row, zeros_nf)
    agg = partials.reshape(NC * HALF, F)[:N]
    out = _node_mlp(h, agg, w3at, w3bt, b3r, w4t, b4r)
    return (out, coord)


# speedup vs baseline: 4.7067x; 1.5495x over previous
import functools

import jax
import jax.numpy as jnp
from jax import lax
from jax.experimental import pallas as pl
from jax.experimental.pallas import tpu as pltpu
from jax.experimental.pallas import tpu_sc as plsc

N = 10000
E = 320000
F = 128

NC = 2
NS = 16
NW = NC * NS
PER_W = E // NW
K = 80
ITERS = PER_W // K
NPAD = 10240
ROWS_PER_TILE = NPAD // NS

BN = 1000
BE = 2560
XR = BE // F


def _node_pre_body(h_ref, c_ref, w1at_ref, w1bt_ref, b1_ref, w1r_ref,
                   a_ref, b_ref):
    h = h_ref[...]
    cc = c_ref[...]
    c2 = jnp.sum(cc * cc, axis=1, keepdims=True)
    rad = c2 * w1r_ref[...]
    a_ref[...] = jnp.dot(h, w1at_ref[...],
                         preferred_element_type=jnp.float32) + b1_ref[...] + rad
    b_ref[...] = jnp.dot(h, w1bt_ref[...],
                         preferred_element_type=jnp.float32) + rad


def _node_pre(h, coord, w1at, w1bt, b1, w1r):
    grid = (N // BN,)
    return pl.pallas_call(
        _node_pre_body,
        grid=grid,
        in_specs=[
            pl.BlockSpec((BN, F), lambda i: (i, 0)),
            pl.BlockSpec((BN, 3), lambda i: (i, 0)),
            pl.BlockSpec((F, F), lambda i: (0, 0)),
            pl.BlockSpec((F, F), lambda i: (0, 0)),
            pl.BlockSpec((1, F), lambda i: (0, 0)),
            pl.BlockSpec((1, F), lambda i: (0, 0)),
        ],
        out_specs=[
            pl.BlockSpec((BN, F), lambda i: (i, 0)),
            pl.BlockSpec((BN, F), lambda i: (i, 0)),
        ],
        out_shape=[
            jax.ShapeDtypeStruct((N, F), jnp.float32),
            jax.ShapeDtypeStruct((N, F), jnp.float32),
        ],
    )(h, coord, w1at, w1bt, b1, w1r)


def _gather_body(ta, tb, c3_hbm, row_hbm, col_hbm,
                 g1_hbm, g2_hbm, cx_hbm,
                 rowall, colall, c3v,
                 av0, bv0, av1, bv1, xv0, xv1,
                 gsem0, gsem1, wsem0, wsem1):
    wid = lax.axis_index("s") * NC + lax.axis_index("c")
    base = pl.multiple_of(wid * PER_W, 8)

    pltpu.sync_copy(row_hbm.at[pl.ds(base, PER_W)], rowall)
    pltpu.sync_copy(col_hbm.at[pl.ds(base, PER_W)], colall)
    pltpu.sync_copy(c3_hbm, c3v)

    zero16 = jnp.zeros((16,), jnp.int32)
    one16 = jnp.full((16,), 1, jnp.int32)
    two16 = jnp.full((16,), 2, jnp.int32)

    def start_g(i, av, bv, sem):
        o = pl.multiple_of(i * K, 8)
        pltpu.async_copy(ta.at[rowall.at[pl.ds(o, K)]], av, sem)
        pltpu.async_copy(tb.at[colall.at[pl.ds(o, K)]], bv, sem)

    def wait_g(av, bv, sem):
        idx0 = rowall.at[pl.ds(0, K)]
        pltpu.make_async_copy(ta.at[idx0], av, sem).wait()
        pltpu.make_async_copy(tb.at[idx0], bv, sem).wait()

    def cross_chunk(i, xv):
        o = pl.multiple_of(i * K, 8)
        for j in range(K // 16):
            ir = rowall[pl.ds(o + j * 16, 16)]
            ic = colall[pl.ds(o + j * 16, 16)]
            xr = plsc.load_gather(c3v, [zero16, ir])
            yr = plsc.load_gather(c3v, [one16, ir])
            zr = plsc.load_gather(c3v, [two16, ir])
            xc = plsc.load_gather(c3v, [zero16, ic])
            yc = plsc.load_gather(c3v, [one16, ic])
            zc = plsc.load_gather(c3v, [two16, ic])
            xv[pl.ds(j * 16, 16)] = xr * xc + yr * yc + zr * zc

    def start_w(i, av, bv, xv, sem):
        off = pl.multiple_of(base + i * K, 8)
        pltpu.async_copy(av, g1_hbm.at[pl.ds(off, K)], sem)
        pltpu.async_copy(bv, g2_hbm.at[pl.ds(off, K)], sem)
        pltpu.async_copy(xv, cx_hbm.at[pl.ds(off, K)], sem)

    def wait_w(av, bv, xv, sem):
        pltpu.make_async_copy(av, g1_hbm.at[pl.ds(0, K)], sem).wait()
        pltpu.make_async_copy(bv, g2_hbm.at[pl.ds(0, K)], sem).wait()
        pltpu.make_async_copy(xv, cx_hbm.at[pl.ds(0, K)], sem).wait()

    start_g(0, av0, bv0, gsem0)

    def body(t, carry):
        c0 = 2 * t

        @pl.when(t > 0)
        def _():
            wait_w(av1, bv1, xv1, wsem1)

        start_g(c0 + 1, av1, bv1, gsem1)
        cross_chunk(c0, xv0)
        wait_g(av0, bv0, gsem0)
        start_w(c0, av0, bv0, xv0, wsem0)
        wait_w(av0, bv0, xv0, wsem0)

        @pl.when(c0 + 2 < ITERS)
        def _():
            start_g(c0 + 2, av0, bv0, gsem0)

        cross_chunk(c0 + 1, xv1)
        wait_g(av1, bv1, gsem1)
        start_w(c0 + 1, av1, bv1, xv1, wsem1)
        return carry

    lax.fori_loop(0, ITERS // 2, body, 0)

    wait_w(av1, bv1, xv1, wsem1)
    cross_chunk(ITERS - 1, xv0)
    wait_g(av0, bv0, gsem0)
    start_w(ITERS - 1, av0, bv0, xv0, wsem0)
    wait_w(av0, bv0, xv0, wsem0)


def _sc_gather(ta, tb, c3, row, col):
    mesh = plsc.VectorSubcoreMesh(core_axis_name="c", subcore_axis_name="s")
    kfn = pl.kernel(
        _gather_body,
        out_type=[
            jax.ShapeDtypeStruct((E, F), jnp.float32),
            jax.ShapeDtypeStruct((E, F), jnp.float32),
            jax.ShapeDtypeStruct((E,), jnp.float32),
        ],
        mesh=mesh,
        scratch_types=[
            pltpu.VMEM((PER_W,), jnp.int32),
            pltpu.VMEM((PER_W,), jnp.int32),
            pltpu.VMEM((3, N), jnp.float32),
            pltpu.VMEM((K, F), jnp.float32),
            pltpu.VMEM((K, F), jnp.float32),
            pltpu.VMEM((K, F), jnp.float32),
            pltpu.VMEM((K, F), jnp.float32),
            pltpu.VMEM((K,), jnp.float32),
            pltpu.VMEM((K,), jnp.float32),
            pltpu.SemaphoreType.DMA,
            pltpu.SemaphoreType.DMA,
            pltpu.SemaphoreType.DMA,
            pltpu.SemaphoreType.DMA,
        ],
        compiler_params=pltpu.CompilerParams(needs_layout_passes=False),
    )
    return kfn(ta, tb, c3, row, col)


def _edge_body(g1_ref, g2_ref, x_ref, ea_ref,
               w1et_ref, w2t_ref, b2_ref, w1rn2_ref, y_ref):
    xb = x_ref[0]
    bcast = jnp.broadcast_to(xb.reshape(XR, 1, F), (XR, F, F)).reshape(BE, F)
    li = lax.broadcasted_iota(jnp.int32, (XR, F, F), 1).reshape(BE, F)
    ji = lax.broadcasted_iota(jnp.int32, (XR, F, F), 2).reshape(BE, F)
    diag = jnp.where(li == ji, bcast, 0.0)
    wfull = jnp.broadcast_to(w1rn2_ref[...], (F, F))
    pre1 = (g1_ref[...] + g2_ref[...]
            + jnp.dot(diag, wfull, preferred_element_type=jnp.float32)
            + jnp.dot(ea_ref[...], w1et_ref[...],
                      preferred_element_type=jnp.float32))
    m = pre1 * jax.nn.sigmoid(pre1)
    pre2 = jnp.dot(m, w2t_ref[...],
                   preferred_element_type=jnp.float32) + b2_ref[...]
    y_ref[...] = pre2 * jax.nn.sigmoid(pre2)


def _edge_mlp(g1, g2, x3, ea, w1et, w2t, b2, w1rn2):
    grid = (E // BE,)
    return pl.pallas_call(
        _edge_body,
        grid=grid,
        in_specs=[
            pl.BlockSpec((BE, F), lambda i: (i, 0)),
            pl.BlockSpec((BE, F), lambda i: (i, 0)),
            pl.BlockSpec((1, XR, F), lambda i: (i, 0, 0)),
            pl.BlockSpec((BE, 4), lambda i: (i, 0)),
            pl.BlockSpec((4, F), lambda i: (0, 0)),
            pl.BlockSpec((F, F), lambda i: (0, 0)),
            pl.BlockSpec((1, F), lambda i: (0, 0)),
            pl.BlockSpec((1, F), lambda i: (0, 0)),
        ],
        out_specs=pl.BlockSpec((BE, F), lambda i: (i, 0)),
        out_shape=jax.ShapeDtypeStruct((E, F), jnp.float32),
    )(g1, g2, x3, ea, w1et, w2t, b2, w1rn2)


HALF = 5120
ACC_ROWS = HALF + 128
ZROWS = ACC_ROWS // NS
PER_TILE_E = E // NS
SC_ITERS = PER_TILE_E // K
DUMP_ROWS = HALF // NS


def _scatter_body(y_hbm, row_hbm, zeros_hbm, out_hbm,
                  it0, it1, ix0, ix1, yv0, yv1, dumpv, acc,
                  lsem0, lsem1, ssem0, ssem1):
    c = lax.axis_index("c")
    s = lax.axis_index("s")
    base = pl.multiple_of(s * PER_TILE_E, 8)
    lo = c * HALF

    pltpu.sync_copy(zeros_hbm, acc.at[pl.ds(s * ZROWS, ZROWS)])

    def start_l(i, it, yv, sem):
        off = pl.multiple_of(base + i * K, 8)
        pltpu.async_copy(row_hbm.at[pl.ds(off, K)], it, sem)
        pltpu.async_copy(y_hbm.at[pl.ds(off, K)], yv, sem)

    def wait_l(it, yv, sem):
        pltpu.make_async_copy(row_hbm.at[pl.ds(0, K)], it, sem).wait()
        pltpu.make_async_copy(y_hbm.at[pl.ds(0, K)], yv, sem).wait()

    def remap(it, ix):
        for j in range(K // 16):
            v = it[pl.ds(j * 16, 16)]
            v2 = v - lo
            in_half = (v2 >= 0) & (v2 < HALF)
            ix[pl.ds(j * 16, 16)] = jnp.where(in_half, v2, HALF + (v & 127))

    def start_s(ix, yv, sem):
        pltpu.async_copy(yv, acc.at[ix], sem, add=True)

    def wait_s(ix, yv, sem):
        pltpu.make_async_copy(yv, acc.at[ix], sem).wait()

    start_l(0, it0, yv0, lsem0)
    plsc.subcore_barrier()

    def body(t, carry):
        c0 = 2 * t
        wait_l(it0, yv0, lsem0)
        remap(it0, ix0)
        start_s(ix0, yv0, ssem0)

        @pl.when(t > 0)
        def _():
            wait_s(ix1, yv1, ssem1)

        start_l(c0 + 1, it1, yv1, lsem1)
        wait_l(it1, yv1, lsem1)
        remap(it1, ix1)
        start_s(ix1, yv1, ssem1)
        wait_s(ix0, yv0, ssem0)

        @pl.when(t + 1 < SC_ITERS // 2)
        def _():
            start_l(c0 + 2, it0, yv0, lsem0)

        return carry

    lax.fori_loop(0, SC_ITERS // 2, body, 0)
    wait_s(ix1, yv1, ssem1)
    plsc.subcore_barrier()

    pltpu.sync_copy(acc.at[pl.ds(s * DUMP_ROWS, DUMP_ROWS)], dumpv)
    pltpu.sync_copy(dumpv, out_hbm.at[c].at[pl.ds(s * DUMP_ROWS, DUMP_ROWS)])


def _sc_scatter(y, row, zeros_nf):
    mesh = plsc.VectorSubcoreMesh(core_axis_name="c", subcore_axis_name="s")
    kfn = pl.kernel(
        _scatter_body,
        out_type=jax.ShapeDtypeStruct((NC, HALF, F), jnp.float32),
        mesh=mesh,
        scratch_types=[
            pltpu.VMEM((K,), jnp.int32),
            pltpu.VMEM((K,), jnp.int32),
            pltpu.VMEM((K,), jnp.int32),
            pltpu.VMEM((K,), jnp.int32),
            pltpu.VMEM((K, F), jnp.float32),
            pltpu.VMEM((K, F), jnp.float32),
            pltpu.VMEM((DUMP_ROWS, F), jnp.float32),
            pltpu.VMEM_SHARED((ACC_ROWS, F), jnp.float32),
            pltpu.SemaphoreType.DMA,
            pltpu.SemaphoreType.DMA,
            pltpu.SemaphoreType.DMA,
            pltpu.SemaphoreType.DMA,
        ],
    )
    return kfn(y, row, zeros_nf)


def _node_body(h_ref, agg_ref, w3at_ref, w3bt_ref, b3_ref,
               w4t_ref, b4_ref, out_ref):
    h = h_ref[...]
    agg = agg_ref[...]
    pre = (jnp.dot(h, w3at_ref[...], preferred_element_type=jnp.float32)
           + jnp.dot(agg, w3bt_ref[...], preferred_element_type=jnp.float32)
           + b3_ref[...])
    n = pre * jax.nn.sigmoid(pre)
    out_ref[...] = h + jnp.dot(n, w4t_ref[...],
                               preferred_element_type=jnp.float32) + b4_ref[...]


def _node_mlp(h, agg, w3at, w3bt, b3, w4t, b4):
    grid = (N // BN,)
    return pl.pallas_call(
        _node_body,
        grid=grid,
        in_specs=[
            pl.BlockSpec((BN, F), lambda i: (i, 0)),
            pl.BlockSpec((BN, F), lambda i: (i, 0)),
            pl.BlockSpec((F, F), lambda i: (0, 0)),
            pl.BlockSpec((F, F), lambda i: (0, 0)),
            pl.BlockSpec((1, F), lambda i: (0, 0)),
            pl.BlockSpec((F, F), lambda i: (0, 0)),
            pl.BlockSpec((1, F), lambda i: (0, 0)),
        ],
        out_specs=pl.BlockSpec((BN, F), lambda i: (i, 0)),
        out_shape=jax.ShapeDtypeStruct((N, F), jnp.float32),
    )(h, agg, w3at, w3bt, b3, w4t, b4)


def kernel(h, coord, edge_index, edge_attr, W1, b1, W2, b2, W3, b3, W4, b4):
    row = edge_index[0].astype(jnp.int32)
    col = edge_index[1].astype(jnp.int32)

    w1at = W1[:, :F].T
    w1bt = W1[:, F:2 * F].T
    w1r = W1[:, 2 * F].reshape(1, F)
    w1et = W1[:, 2 * F + 1:].T
    b1r = b1.reshape(1, F)
    b2r = b2.reshape(1, F)
    w3at = W3[:, :F].T
    w3bt = W3[:, F:].T
    b3r = b3.reshape(1, F)
    w4t = W4.T
    b4r = b4.reshape(1, F)

    c3 = coord.T

    a_tab, b_tab = _node_pre(h, coord, w1at, w1bt, b1r, w1r)
    g1, g2, cross = _sc_gather(a_tab, b_tab, c3, row, col)
    x3 = cross.reshape(E // BE, XR, F)
    y = _edge_mlp(g1, g2, x3, edge_attr, w1et, w2t=W2.T, b2=b2r,
                  w1rn2=-2.0 * w1r)
    zeros_nf = jnp.zeros((ZROWS, F), jnp.float32)
    partials = _sc_scatter(y, row, zeros_nf)
    agg = partials.reshape(NC * HALF, F)[:N]
    out = _node_mlp(h, agg, w3at, w3bt, b3r, w4t, b4r)
    return (out, coord)
